# baseline (speedup 1.0000x reference)
import functools

import jax
import jax.numpy as jnp
from jax import lax
from jax.experimental import pallas as pl
from jax.experimental.pallas import tpu as pltpu
from jax.experimental.pallas import tpu_sc as plsc

NB = 1000
RANK = 16
BATCH = 16384
L = 16

_NC = 2
_NS = 16
_NW = _NC * _NS
_BPW = BATCH // _NW
_STD_PAD = 1024


def _body(cf_hbm, std_hbm, x_hbm, y_hbm, out_hbm, tab_v, std_v, x_v, y_v, o_v,
          sem):
    wid = lax.axis_index("s") * _NC + lax.axis_index("c")
    base = wid * _BPW

    c1 = pltpu.async_copy(cf_hbm, tab_v, sem)
    c2 = pltpu.async_copy(std_hbm, std_v, sem)
    c3 = pltpu.async_copy(x_hbm.at[pl.ds(base, _BPW)], x_v, sem)
    c4 = pltpu.async_copy(y_hbm.at[pl.ds(base, _BPW)], y_v, sem)
    c1.wait()
    c2.wait()
    c3.wait()
    c4.wait()

    lane = lax.iota(jnp.int32, L)
    @plsc.parallel_loop(0, _BPW, step=L, unroll=4)
    def chunk_body(off):
        xv = x_v[pl.ds(off, L)]
        yv = y_v[pl.ds(off, L)]
        xo_vec = xv << 4
        yo_vec = yv << 4
        acc = jnp.zeros((L,), jnp.float32)
        for j in range(L):
            xrow = tab_v[pl.ds(xo_vec[j], L)]
            yrow = tab_v[pl.ds(yo_vec[j], L)]
            t = xrow * yrow
            pair_sum = jnp.sum(t * t)
            acc = jnp.where(lane == j, jnp.full((L,), pair_sum, jnp.float32),
                            acc)
        s = plsc.load_gather(std_v, [xv])
        diag = jnp.where(xv == yv, s * s, jnp.zeros((L,), jnp.float32))
        o_v[pl.ds(off, L)] = acc + diag

    pltpu.sync_copy(o_v, out_hbm.at[pl.ds(base, _BPW)])


def kernel(x, y, sqrt_covar_factor, std):
    cf_flat = sqrt_covar_factor.reshape(-1)
    std_pad = jnp.zeros((_STD_PAD,), jnp.float32).at[:NB].set(std)
    mesh = plsc.VectorSubcoreMesh(core_axis_name="c", subcore_axis_name="s")
    run = functools.partial(
        pl.kernel,
        mesh=mesh,
        compiler_params=pltpu.CompilerParams(needs_layout_passes=False),
        out_type=jax.ShapeDtypeStruct((BATCH,), jnp.float32),
        scratch_types=[
            pltpu.VMEM((NB * RANK,), jnp.float32),
            pltpu.VMEM((_STD_PAD,), jnp.float32),
            pltpu.VMEM((_BPW,), jnp.int32),
            pltpu.VMEM((_BPW,), jnp.int32),
            pltpu.VMEM((_BPW,), jnp.float32),
            pltpu.SemaphoreType.DMA,
        ],
    )(_body)
    return run(cf_flat, std_pad, x, y)


# speedup vs baseline: 1.0012x; 1.0012x over previous
import functools

import jax
import jax.numpy as jnp
from jax import lax
from jax.experimental import pallas as pl
from jax.experimental.pallas import tpu as pltpu
from jax.experimental.pallas import tpu_sc as plsc

NB = 1000
RANK = 16
BATCH = 16384
L = 16

_NC = 2
_NS = 16
_NW = _NC * _NS
_BPW = BATCH // _NW
_STD_PAD = 1024


def _body(cf_hbm, std_hbm, x_hbm, y_hbm, out_hbm, tab_v, std_v, x_v, y_v, o_v,
          sem):
    wid = lax.axis_index("s") * _NC + lax.axis_index("c")
    base = wid * _BPW

    c1 = pltpu.async_copy(cf_hbm, tab_v, sem)
    c2 = pltpu.async_copy(std_hbm, std_v.at[pl.ds(0, NB)], sem)
    c3 = pltpu.async_copy(x_hbm.at[pl.ds(base, _BPW)], x_v, sem)
    c4 = pltpu.async_copy(y_hbm.at[pl.ds(base, _BPW)], y_v, sem)
    c1.wait()
    c2.wait()
    c3.wait()
    c4.wait()

    lane = lax.iota(jnp.int32, L)
    @plsc.parallel_loop(0, _BPW, step=L, unroll=2)
    def chunk_body(off):
        xv = x_v[pl.ds(off, L)]
        yv = y_v[pl.ds(off, L)]
        xo_vec = xv << 4
        yo_vec = yv << 4
        acc = jnp.zeros((L,), jnp.float32)
        for j in range(L):
            xrow = tab_v[pl.ds(xo_vec[j], L)]
            yrow = tab_v[pl.ds(yo_vec[j], L)]
            t = xrow * yrow
            pair_sum = jnp.sum(t * t)
            acc = jnp.where(lane == j, jnp.full((L,), pair_sum, jnp.float32),
                            acc)
        s = plsc.load_gather(std_v, [xv])
        diag = jnp.where(xv == yv, s * s, jnp.zeros((L,), jnp.float32))
        o_v[pl.ds(off, L)] = acc + diag

    pltpu.sync_copy(o_v, out_hbm.at[pl.ds(base, _BPW)])


def kernel(x, y, sqrt_covar_factor, std):
    cf_flat = sqrt_covar_factor.reshape(-1)
    mesh = plsc.VectorSubcoreMesh(core_axis_name="c", subcore_axis_name="s")
    run = functools.partial(
        pl.kernel,
        mesh=mesh,
        compiler_params=pltpu.CompilerParams(needs_layout_passes=False),
        out_type=jax.ShapeDtypeStruct((BATCH,), jnp.float32),
        scratch_types=[
            pltpu.VMEM((NB * RANK,), jnp.float32),
            pltpu.VMEM((_STD_PAD,), jnp.float32),
            pltpu.VMEM((_BPW,), jnp.int32),
            pltpu.VMEM((_BPW,), jnp.int32),
            pltpu.VMEM((_BPW,), jnp.float32),
            pltpu.SemaphoreType.DMA,
        ],
    )(_body)
    return run(cf_flat, std, x, y)


# speedup vs baseline: 1.0126x; 1.0113x over previous
import functools

import jax
import jax.numpy as jnp
from jax import lax
from jax.experimental import pallas as pl
from jax.experimental.pallas import tpu as pltpu
from jax.experimental.pallas import tpu_sc as plsc

NB = 1000
RANK = 16
BATCH = 16384
L = 16

_NC = 2
_NS = 16
_NW = _NC * _NS
_BPW = BATCH // _NW
_STD_PAD = 1024


def _body(cf_hbm, std_hbm, x_hbm, y_hbm, out_hbm, tab_v, std_v, x_v, y_v, o_v,
          sem):
    wid = lax.axis_index("s") * _NC + lax.axis_index("c")
    base = wid * _BPW

    c1 = pltpu.async_copy(cf_hbm, tab_v, sem)
    c2 = pltpu.async_copy(std_hbm, std_v, sem)
    c3 = pltpu.async_copy(x_hbm.at[pl.ds(base, _BPW)], x_v, sem)
    c4 = pltpu.async_copy(y_hbm.at[pl.ds(base, _BPW)], y_v, sem)
    c1.wait()
    c2.wait()
    c3.wait()
    c4.wait()

    lane = lax.iota(jnp.int32, L)
    @plsc.parallel_loop(0, _BPW, step=L, unroll=2)
    def chunk_body(off):
        xv = x_v[pl.ds(off, L)]
        yv = y_v[pl.ds(off, L)]
        xo_vec = xv << 4
        yo_vec = yv << 4
        acc = jnp.zeros((L,), jnp.float32)
        for j in range(L):
            xrow = tab_v[pl.ds(xo_vec[j], L)]
            yrow = tab_v[pl.ds(yo_vec[j], L)]
            t = xrow * yrow
            pair_sum = jnp.sum(t * t)
            acc = jnp.where(lane == j, jnp.full((L,), pair_sum, jnp.float32),
                            acc)
        s = plsc.load_gather(std_v, [xv])
        diag = jnp.where(xv == yv, s * s, jnp.zeros((L,), jnp.float32))
        o_v[pl.ds(off, L)] = acc + diag

    pltpu.sync_copy(o_v, out_hbm.at[pl.ds(base, _BPW)])


def kernel(x, y, sqrt_covar_factor, std):
    cf_flat = sqrt_covar_factor.reshape(-1)
    std_pad = jnp.zeros((_STD_PAD,), jnp.float32).at[:NB].set(std)
    mesh = plsc.VectorSubcoreMesh(core_axis_name="c", subcore_axis_name="s")
    run = functools.partial(
        pl.kernel,
        mesh=mesh,
        compiler_params=pltpu.CompilerParams(needs_layout_passes=False),
        out_type=jax.ShapeDtypeStruct((BATCH,), jnp.float32),
        scratch_types=[
            pltpu.VMEM((NB * RANK,), jnp.float32),
            pltpu.VMEM((_STD_PAD,), jnp.float32),
            pltpu.VMEM((_BPW,), jnp.int32),
            pltpu.VMEM((_BPW,), jnp.int32),
            pltpu.VMEM((_BPW,), jnp.float32),
            pltpu.SemaphoreType.DMA,
        ],
    )(_body)
    return run(cf_flat, std_pad, x, y)
